# baseline (speedup 1.0000x reference)
import functools

import jax
import jax.numpy as jnp
from jax import lax
from jax.experimental import pallas as pl
from jax.experimental.pallas import tpu as pltpu
from jax.experimental.pallas import tpu_sc as plsc

_VOCAB = 100000
_EMBED_DIM = 64
_BATCH = 4096
_SEQ = 3
_B = _BATCH * _SEQ

_NUM_CORES = 2
_NUM_SUBCORES = 16
_NW = _NUM_CORES * _NUM_SUBCORES
_LANES = 128
_B_PER_W = _SEQ * _LANES
_CHUNK = 128


_TBLK = 4096
_N_TBLOCKS = (_VOCAB + _TBLK - 1) // _TBLK


def _transpose_body(t_ref, o_ref):
    t2 = t_ref[...].T
    pieces = [
        jnp.concatenate(
            [t2[128 * g:128 * g + _EMBED_DIM, :],
             t2[128 * g + _EMBED_DIM:128 * (g + 1), :]], axis=1)
        for g in range(_TBLK // 128)
    ]
    o_ref[...] = jnp.concatenate(pieces, axis=0)


def _pack_pairs(table_t):
    return pl.pallas_call(
        _transpose_body,
        grid=(_N_TBLOCKS,),
        in_specs=[pl.BlockSpec((_EMBED_DIM, _TBLK), lambda c: (0, c))],
        out_specs=pl.BlockSpec((_TBLK // 2, 2 * _EMBED_DIM), lambda c: (c, 0)),
        out_shape=jax.ShapeDtypeStruct(
            (_N_TBLOCKS * _TBLK // 2, 2 * _EMBED_DIM), jnp.float32),
    )(table_t)


def _gather_body(tab_hbm, idx_hbm, out_hbm, idx_vm, line_vm, col_vm, rows_vm,
                 outbuf, sem, osem):
    wid = lax.axis_index("s") * _NUM_CORES + lax.axis_index("c")
    pltpu.sync_copy(idx_hbm.at[wid], idx_vm)
    for s in range(_SEQ):
        for c in range(_LANES // 16):
            w = idx_vm[s, pl.ds(16 * c, 16)]
            off = s * _LANES + 16 * c
            line_vm[pl.ds(off, 16)] = lax.shift_left(
                lax.shift_right_logical(w, 7), 6) + lax.bitwise_and(w, 63)
            col_vm[pl.ds(off, 16)] = lax.bitwise_and(w, _EMBED_DIM)
    copies = [
        pltpu.async_copy(
            tab_hbm.at[line_vm.at[pl.ds(s * _CHUNK, _CHUNK)]],
            rows_vm.at[pl.ds(s * _CHUNK, _CHUNK), :],
            sem,
        )
        for s in range(_SEQ)
    ]
    out_copies = []
    for s in range(_SEQ):
        copies[s].wait()
        for c in range(_LANES // 16):
            row_idx = jnp.full((16,), s * _LANES + 16 * c, jnp.int32) + lax.iota(
                jnp.int32, 16)
            col_base = col_vm[pl.ds(s * _LANES + 16 * c, 16)]
            for e0 in range(0, _EMBED_DIM, 8):
                gs = [
                    plsc.load_gather(rows_vm, [row_idx, col_base + (e0 + k)])
                    for k in range(8)
                ]
                for k in range(8):
                    outbuf[s * 8 + (e0 + k) // 8, (e0 + k) % 8,
                           pl.ds(16 * c, 16)] = gs[k]
        for et in range(_EMBED_DIM // 8):
            out_copies.append(
                pltpu.async_copy(
                    outbuf.at[s * 8 + et], out_hbm.at[s, et, wid], osem))
    for cp in out_copies:
        cp.wait()


def _encoder_gather(idxp, tab_pad):
    mesh = plsc.VectorSubcoreMesh(core_axis_name="c", subcore_axis_name="s")
    k = functools.partial(
        pl.kernel,
        mesh=mesh,
        out_type=jax.ShapeDtypeStruct(
            (_SEQ, _EMBED_DIM // 8, _NW, 8, _LANES), jnp.float32),
        scratch_types=[
            pltpu.VMEM((8, _LANES), jnp.int32),
            pltpu.VMEM((_B_PER_W,), jnp.int32),
            pltpu.VMEM((_B_PER_W,), jnp.int32),
            pltpu.VMEM((_B_PER_W, 2 * _EMBED_DIM), jnp.float32),
            pltpu.VMEM((_SEQ * (_EMBED_DIM // 8), 8, _LANES), jnp.float32),
            pltpu.SemaphoreType.DMA,
            pltpu.SemaphoreType.DMA,
        ],
        compiler_params=pltpu.CompilerParams(needs_layout_passes=False),
    )(_gather_body)
    return k(tab_pad, idxp)


def kernel(x, table):
    tab_pairs = _pack_pairs(table.T)
    xr = jnp.transpose(x.reshape(_NW, _LANES, _SEQ), (0, 2, 1))
    idxp = jnp.pad(xr, ((0, 0), (0, 8 - _SEQ), (0, 0)))
    out5 = _encoder_gather(idxp, tab_pairs)
    out = jnp.transpose(out5, (2, 4, 0, 1, 3)).reshape(_BATCH, _SEQ, _EMBED_DIM)
    return out


# speedup vs baseline: 1.0898x; 1.0898x over previous
import functools

import jax
import jax.numpy as jnp
from jax import lax
from jax.experimental import pallas as pl
from jax.experimental.pallas import tpu as pltpu
from jax.experimental.pallas import tpu_sc as plsc

_VOCAB = 100000
_EMBED_DIM = 64
_BATCH = 4096
_SEQ = 3
_B = _BATCH * _SEQ

_NUM_CORES = 2
_NUM_SUBCORES = 16
_NW = _NUM_CORES * _NUM_SUBCORES
_LANES = 128
_B_PER_W = _SEQ * _LANES
_CHUNK = 64


_TBLK = 8192
_N_TBLOCKS = (_VOCAB + _TBLK - 1) // _TBLK


def _transpose_body(t_ref, o_ref):
    t2 = t_ref[...].T
    pieces = [
        jnp.concatenate(
            [t2[128 * g:128 * g + _EMBED_DIM, :],
             t2[128 * g + _EMBED_DIM:128 * (g + 1), :]], axis=1)
        for g in range(_TBLK // 128)
    ]
    o_ref[...] = jnp.concatenate(pieces, axis=0)


def _pack_pairs(table_t):
    return pl.pallas_call(
        _transpose_body,
        grid=(_N_TBLOCKS,),
        in_specs=[pl.BlockSpec((_EMBED_DIM, _TBLK), lambda c: (0, c))],
        out_specs=pl.BlockSpec((_TBLK // 2, 2 * _EMBED_DIM), lambda c: (c, 0)),
        out_shape=jax.ShapeDtypeStruct(
            (_N_TBLOCKS * _TBLK // 2, 2 * _EMBED_DIM), jnp.float32),
    )(table_t)


def _gather_body(tab_hbm, idx_hbm, out_hbm, idx_vm, line_vm, col_vm, rows_vm,
                 outbuf, sem, osem):
    wid = lax.axis_index("s") * _NUM_CORES + lax.axis_index("c")
    pltpu.sync_copy(idx_hbm.at[wid], idx_vm)
    for s in range(_SEQ):
        for c in range(_LANES // 16):
            w = idx_vm[s, pl.ds(16 * c, 16)]
            off = s * _LANES + 16 * c
            line_vm[pl.ds(off, 16)] = lax.shift_left(
                lax.shift_right_logical(w, 7), 6) + lax.bitwise_and(w, 63)
            col_vm[pl.ds(off, 16)] = lax.bitwise_and(w, _EMBED_DIM)
    copies = [
        pltpu.async_copy(
            tab_hbm.at[line_vm.at[pl.ds(k * _CHUNK, _CHUNK)]],
            rows_vm.at[pl.ds(k * _CHUNK, _CHUNK), :],
            sem,
        )
        for k in range(_B_PER_W // _CHUNK)
    ]
    out_copies = []
    per_s = _LANES // _CHUNK
    for s in range(_SEQ):
        for k in range(per_s):
            copies[s * per_s + k].wait()
        for c in range(_LANES // 16):
            row_idx = jnp.full((16,), s * _LANES + 16 * c, jnp.int32) + lax.iota(
                jnp.int32, 16)
            col_base = col_vm[pl.ds(s * _LANES + 16 * c, 16)]
            for e0 in range(0, _EMBED_DIM, 8):
                gs = [
                    plsc.load_gather(rows_vm, [row_idx, col_base + (e0 + k)])
                    for k in range(8)
                ]
                for k in range(8):
                    outbuf[s * 8 + (e0 + k) // 8, (e0 + k) % 8,
                           pl.ds(16 * c, 16)] = gs[k]
        for et in range(_EMBED_DIM // 8):
            out_copies.append(
                pltpu.async_copy(
                    outbuf.at[s * 8 + et], out_hbm.at[s, et, wid], osem))
    for cp in out_copies:
        cp.wait()


def _encoder_gather(idxp, tab_pad):
    mesh = plsc.VectorSubcoreMesh(core_axis_name="c", subcore_axis_name="s")
    k = functools.partial(
        pl.kernel,
        mesh=mesh,
        out_type=jax.ShapeDtypeStruct(
            (_SEQ, _EMBED_DIM // 8, _NW, 8, _LANES), jnp.float32),
        scratch_types=[
            pltpu.VMEM((8, _LANES), jnp.int32),
            pltpu.VMEM((_B_PER_W,), jnp.int32),
            pltpu.VMEM((_B_PER_W,), jnp.int32),
            pltpu.VMEM((_B_PER_W, 2 * _EMBED_DIM), jnp.float32),
            pltpu.VMEM((_SEQ * (_EMBED_DIM // 8), 8, _LANES), jnp.float32),
            pltpu.SemaphoreType.DMA,
            pltpu.SemaphoreType.DMA,
        ],
        compiler_params=pltpu.CompilerParams(needs_layout_passes=False),
    )(_gather_body)
    return k(tab_pad, idxp)


def kernel(x, table):
    tab_pairs = _pack_pairs(table.T)
    xr = jnp.transpose(x.reshape(_NW, _LANES, _SEQ), (0, 2, 1))
    idxp = jnp.pad(xr, ((0, 0), (0, 8 - _SEQ), (0, 0)))
    out5 = _encoder_gather(idxp, tab_pairs)
    out = jnp.transpose(out5, (2, 4, 0, 1, 3)).reshape(_BATCH, _SEQ, _EMBED_DIM)
    return out


# speedup vs baseline: 1.1002x; 1.0096x over previous
import functools

import jax
import jax.numpy as jnp
from jax import lax
from jax.experimental import pallas as pl
from jax.experimental.pallas import tpu as pltpu
from jax.experimental.pallas import tpu_sc as plsc

_VOCAB = 100000
_EMBED_DIM = 64
_BATCH = 4096
_SEQ = 3
_B = _BATCH * _SEQ

_NUM_CORES = 2
_NUM_SUBCORES = 16
_NW = _NUM_CORES * _NUM_SUBCORES
_LANES = 128
_B_PER_W = _SEQ * _LANES
_CHUNK = 32


_TBLK = 8192
_N_TBLOCKS = (_VOCAB + _TBLK - 1) // _TBLK


def _transpose_body(t_ref, o_ref):
    t2 = t_ref[...].T
    pieces = [
        jnp.concatenate(
            [t2[128 * g:128 * g + _EMBED_DIM, :],
             t2[128 * g + _EMBED_DIM:128 * (g + 1), :]], axis=1)
        for g in range(_TBLK // 128)
    ]
    o_ref[...] = jnp.concatenate(pieces, axis=0)


def _pack_pairs(table_t):
    return pl.pallas_call(
        _transpose_body,
        grid=(_N_TBLOCKS,),
        in_specs=[pl.BlockSpec((_EMBED_DIM, _TBLK), lambda c: (0, c))],
        out_specs=pl.BlockSpec((_TBLK // 2, 2 * _EMBED_DIM), lambda c: (c, 0)),
        out_shape=jax.ShapeDtypeStruct(
            (_N_TBLOCKS * _TBLK // 2, 2 * _EMBED_DIM), jnp.float32),
    )(table_t)


def _gather_body(tab_hbm, idx_hbm, out_hbm, idx_vm, line_vm, col_vm, rows_vm,
                 outbuf, sem, osem):
    wid = lax.axis_index("s") * _NUM_CORES + lax.axis_index("c")
    pltpu.sync_copy(idx_hbm.at[wid], idx_vm)
    for s in range(_SEQ):
        for c in range(_LANES // 16):
            w = idx_vm[s, pl.ds(16 * c, 16)]
            off = s * _LANES + 16 * c
            line_vm[pl.ds(off, 16)] = lax.shift_left(
                lax.shift_right_logical(w, 7), 6) + lax.bitwise_and(w, 63)
            col_vm[pl.ds(off, 16)] = lax.bitwise_and(w, _EMBED_DIM)
    copies = [
        pltpu.async_copy(
            tab_hbm.at[line_vm.at[pl.ds(k * _CHUNK, _CHUNK)]],
            rows_vm.at[pl.ds(k * _CHUNK, _CHUNK), :],
            sem,
        )
        for k in range(_B_PER_W // _CHUNK)
    ]
    out_copies = []
    per_s = _LANES // _CHUNK
    for k in range(_B_PER_W // _CHUNK):
        copies[k].wait()
        s = (k * _CHUNK) // _LANES
        for c2 in range(_CHUNK // 16):
            off = k * _CHUNK + 16 * c2
            c = (off % _LANES) // 16
            row_idx = jnp.full((16,), off, jnp.int32) + lax.iota(jnp.int32, 16)
            col_base = col_vm[pl.ds(off, 16)]
            for e0 in range(0, _EMBED_DIM, 8):
                gs = [
                    plsc.load_gather(rows_vm, [row_idx, col_base + (e0 + g)])
                    for g in range(8)
                ]
                for g in range(8):
                    outbuf[s * 8 + (e0 + g) // 8, (e0 + g) % 8,
                           pl.ds(16 * c, 16)] = gs[g]
        if (k + 1) % per_s == 0:
            for et in range(_EMBED_DIM // 8):
                out_copies.append(
                    pltpu.async_copy(
                        outbuf.at[s * 8 + et], out_hbm.at[s, et, wid], osem))
    for cp in out_copies:
        cp.wait()


def _encoder_gather(idxp, tab_pad):
    mesh = plsc.VectorSubcoreMesh(core_axis_name="c", subcore_axis_name="s")
    k = functools.partial(
        pl.kernel,
        mesh=mesh,
        out_type=jax.ShapeDtypeStruct(
            (_SEQ, _EMBED_DIM // 8, _NW, 8, _LANES), jnp.float32),
        scratch_types=[
            pltpu.VMEM((8, _LANES), jnp.int32),
            pltpu.VMEM((_B_PER_W,), jnp.int32),
            pltpu.VMEM((_B_PER_W,), jnp.int32),
            pltpu.VMEM((_B_PER_W, 2 * _EMBED_DIM), jnp.float32),
            pltpu.VMEM((_SEQ * (_EMBED_DIM // 8), 8, _LANES), jnp.float32),
            pltpu.SemaphoreType.DMA,
            pltpu.SemaphoreType.DMA,
        ],
        compiler_params=pltpu.CompilerParams(needs_layout_passes=False),
    )(_gather_body)
    return k(tab_pad, idxp)


def kernel(x, table):
    tab_pairs = _pack_pairs(table.T)
    xr = jnp.transpose(x.reshape(_NW, _LANES, _SEQ), (0, 2, 1))
    idxp = jnp.pad(xr, ((0, 0), (0, 8 - _SEQ), (0, 0)))
    out5 = _encoder_gather(idxp, tab_pairs)
    out = jnp.transpose(out5, (2, 4, 0, 1, 3)).reshape(_BATCH, _SEQ, _EMBED_DIM)
    return out
